# baseline (speedup 1.0000x reference)
import jax
import jax.numpy as jnp
from jax.experimental import pallas as pl

_B, _C, _N, _K, _M, _H = 4, 128, 2048, 32, 1024, 4
_DEPTH = _C // _H
_RB = 256
_NB = 128


def _pairwise_kernel(xr_ref, xf_ref, o_ref):
    xr = xr_ref[0]
    xf = xf_ref[0]
    d = jax.lax.dot_general(xr, xf, (((0,), (0,)), ((), ())),
                            preferred_element_type=jnp.float32)
    inner = -2.0 * d
    xx_f = jnp.sum(xf * xf, axis=0, keepdims=True)
    xx_r = jnp.sum(xr * xr, axis=0, keepdims=True)
    t1 = -xx_r.reshape(_RB, 1)
    o_ref[0] = (t1 - inner) - xx_f


def _pairwise(x):
    return pl.pallas_call(
        _pairwise_kernel,
        grid=(_B, _N // _RB),
        in_specs=[
            pl.BlockSpec((1, _C, _RB), lambda b, j: (b, 0, j)),
            pl.BlockSpec((1, _C, _N), lambda b, j: (b, 0, 0)),
        ],
        out_specs=pl.BlockSpec((1, _RB, _N), lambda b, j: (b, j, 0)),
        out_shape=jax.ShapeDtypeStruct((_B, _N, _N), jnp.float32),
    )(x, x)


_INV_SQRT_D = float(jnp.float32(1.0) / jnp.sqrt(jnp.float32(_DEPTH)))
_INV_K = float(jnp.float32(1.0) / jnp.float32(_K))


def _attn_kernel(xb_ref, xf_ref, idxt_ref, wq_ref, wk_ref, attn_ref, aps_ref):
    xb = xb_ref[0]
    xf = xf_ref[0]
    idxt = idxt_ref[0]
    wq = wq_ref[...]
    wk = wk_ref[...]

    slabs = []
    for kk in range(_K):
        ids = idxt[kk:kk + 1, :]
        row = jax.lax.broadcasted_iota(jnp.int32, (_N, _NB), 0)
        oh = jnp.where(row == ids, 1.0, 0.0).astype(jnp.float32)
        slabs.append(jax.lax.dot_general(
            xf, oh, (((1,), (0,)), ((), ())),
            precision=jax.lax.Precision.HIGHEST,
            preferred_element_type=jnp.float32))
    neigh = jnp.concatenate(slabs, axis=1)
    center = jnp.concatenate([xb] * _K, axis=1)
    diff = neigh - center

    k = jax.lax.dot_general(wk, diff, (((1,), (0,)), ((), ())),
                            preferred_element_type=jnp.float32)
    q = jax.lax.dot_general(wq, xb, (((1,), (0,)), ((), ())),
                            preferred_element_type=jnp.float32)
    qrep = jnp.concatenate([q] * _K, axis=1)
    prod = qrep * k
    e = jnp.sum(prod.reshape(_H, _DEPTH, _K * _NB), axis=1)
    e = e.reshape(_H, _K, _NB)
    et = jnp.transpose(e, (0, 2, 1))

    es = et * jnp.float32(_INV_SQRT_D)
    m = jnp.max(es, axis=-1, keepdims=True)
    p = jnp.exp(es - m)
    s = jnp.sum(p, axis=-1, keepdims=True)
    attn = p / s
    mean = jnp.sum(attn, axis=-1, keepdims=True) * jnp.float32(_INV_K)
    dev = attn - mean
    var = jnp.sum(dev * dev, axis=-1) * jnp.float32(_INV_K)
    aps = jnp.sqrt(var)
    attn_ref[0] = attn
    aps_ref[0] = aps


def _attention(x, idx_t, Wq, Wk):
    return pl.pallas_call(
        _attn_kernel,
        grid=(_B, _N // _NB),
        in_specs=[
            pl.BlockSpec((1, _C, _NB), lambda b, j: (b, 0, j)),
            pl.BlockSpec((1, _C, _N), lambda b, j: (b, 0, 0)),
            pl.BlockSpec((1, _K, _NB), lambda b, j: (b, 0, j)),
            pl.BlockSpec((_C, _C), lambda b, j: (0, 0)),
            pl.BlockSpec((_C, _C), lambda b, j: (0, 0)),
        ],
        out_specs=[
            pl.BlockSpec((1, _H, _NB, _K), lambda b, j: (b, 0, j, 0)),
            pl.BlockSpec((1, _H, _NB), lambda b, j: (b, 0, j)),
        ],
        out_shape=[
            jax.ShapeDtypeStruct((_B, _H, _N, _K), jnp.float32),
            jax.ShapeDtypeStruct((_B, _H, _N), jnp.float32),
        ],
    )(x, x, idx_t, Wq, Wk)


def _split_heads(t):
    b, c, n, l = t.shape
    t = t.reshape(b, _H, c // _H, n, l)
    return jnp.transpose(t, (0, 1, 3, 4, 2))


def kernel(x, Wq, Wk, Wv):
    B, C, N, K, M, H = _B, _C, _N, _K, _M, _H
    DEPTH = _DEPTH
    pairwise = _pairwise(x)
    _, idx_nn = jax.lax.top_k(pairwise, K)

    idx_t = jnp.transpose(idx_nn, (0, 2, 1)).astype(jnp.int32)
    attn_bhnk, aps = _attention(x, idx_t, Wq, Wk)
    attn = attn_bhnk[:, :, :, None, :]

    neigh = jax.vmap(lambda xb, ib: xb[:, ib])(x, idx_nn)
    diff = neigh - x[:, :, :, None]
    v = jnp.einsum('oc,bcnk->bonk', Wv, diff)
    v = _split_heads(v)
    _, idx_top = jax.lax.top_k(aps, M)
    _, idx_drop = jax.lax.top_k(-aps, N - M)

    def _gather(att, vv, idx, m):
        ia = jnp.broadcast_to(idx[:, :, :, None, None], (B, H, m, 1, K))
        a_sel = jnp.take_along_axis(att, ia, axis=2)
        iv = jnp.broadcast_to(idx[:, :, :, None, None], (B, H, m, K, DEPTH))
        v_sel = jnp.take_along_axis(vv, iv, axis=2)
        out = jnp.einsum('bhmlk,bhmkd->bhmld', a_sel, v_sel)[:, :, :, 0, :]
        out = jnp.transpose(out, (0, 2, 1, 3)).reshape(B, m, H * DEPTH)
        return jnp.transpose(out, (0, 2, 1))

    x_ds = _gather(attn, v, idx_top, M)
    x_drop = _gather(attn, v, idx_drop, N - M)
    return ((x_ds, idx_top), (x_drop, idx_drop))


# speedup vs baseline: 1.0510x; 1.0510x over previous
import jax
import jax.numpy as jnp
import numpy as np
from jax.experimental import pallas as pl

_B, _C, _N, _K, _M, _H = 4, 128, 2048, 32, 1024, 4
_DEPTH = _C // _H
_RB = 256
_NB = 128


def _pairwise_kernel(xr_ref, xf_ref, o_ref):
    xr = xr_ref[0]
    xf = xf_ref[0]
    d = jax.lax.dot_general(xr, xf, (((0,), (0,)), ((), ())),
                            preferred_element_type=jnp.float32)
    inner = -2.0 * d
    xx_f = jnp.sum(xf * xf, axis=0, keepdims=True)
    xx_r = jnp.sum(xr * xr, axis=0, keepdims=True)
    t1 = -xx_r.reshape(_RB, 1)
    o_ref[0] = (t1 - inner) - xx_f


def _pairwise(x):
    return pl.pallas_call(
        _pairwise_kernel,
        grid=(_B, _N // _RB),
        in_specs=[
            pl.BlockSpec((1, _C, _RB), lambda b, j: (b, 0, j)),
            pl.BlockSpec((1, _C, _N), lambda b, j: (b, 0, 0)),
        ],
        out_specs=pl.BlockSpec((1, _RB, _N), lambda b, j: (b, j, 0)),
        out_shape=jax.ShapeDtypeStruct((_B, _N, _N), jnp.float32),
    )(x, x)


def _knn_kernel(xr_ref, xf_ref, idx_ref):
    xr = xr_ref[0]
    xf = xf_ref[0]
    d = jax.lax.dot_general(xr, xf, (((0,), (0,)), ((), ())),
                            preferred_element_type=jnp.float32)
    inner = -2.0 * d
    xx_f = jnp.sum(xf * xf, axis=0, keepdims=True)
    xx_r = jnp.sum(xr * xr, axis=0, keepdims=True)
    t1 = -xx_r.reshape(_RB, 1)
    p = (t1 - inner) - xx_f

    lane = jax.lax.broadcasted_iota(jnp.int32, (_RB, _N), 1)
    col = jax.lax.broadcasted_iota(jnp.int32, (_RB, _K), 1)

    def body(kk, carry):
        p, acc = carry
        m = jnp.max(p, axis=1, keepdims=True)
        cand = jnp.where(p == m, lane, _N)
        amin = jnp.min(cand, axis=1, keepdims=True)
        acc = jnp.where(col == kk, amin, acc)
        p = jnp.where(lane == amin, -jnp.inf, p)
        return p, acc

    _, acc = jax.lax.fori_loop(
        0, _K, body, (p, jnp.zeros((_RB, _K), jnp.int32)))
    idx_ref[0] = acc


def _knn(x):
    return pl.pallas_call(
        _knn_kernel,
        grid=(_B, _N // _RB),
        in_specs=[
            pl.BlockSpec((1, _C, _RB), lambda b, j: (b, 0, j)),
            pl.BlockSpec((1, _C, _N), lambda b, j: (b, 0, 0)),
        ],
        out_specs=pl.BlockSpec((1, _RB, _K), lambda b, j: (b, j, 0)),
        out_shape=jax.ShapeDtypeStruct((_B, _N, _K), jnp.int32),
    )(x, x)


_INV_SQRT_D = float(np.float32(1.0) / np.sqrt(np.float32(_DEPTH)))
_INV_K = float(np.float32(1.0) / np.float32(_K))
_LOG2E = float(np.float32(1.4426950408889634))


def _attn_kernel(xb_ref, xf_ref, idxt_ref, wq_ref, wk_ref, attn_ref, aps_ref):
    xb = xb_ref[0]
    xf = xf_ref[0]
    idxt = idxt_ref[0]
    wq = wq_ref[...]
    wk = wk_ref[...]

    slabs = []
    for kk in range(_K):
        ids = idxt[kk:kk + 1, :]
        row = jax.lax.broadcasted_iota(jnp.int32, (_N, _NB), 0)
        oh = jnp.where(row == ids, 1.0, 0.0).astype(jnp.float32)
        slabs.append(jax.lax.dot_general(
            xf, oh, (((1,), (0,)), ((), ())),
            precision=jax.lax.Precision.HIGHEST,
            preferred_element_type=jnp.float32))
    neigh = jnp.concatenate(slabs, axis=1)
    center = jnp.concatenate([xb] * _K, axis=1)
    diff = neigh - center

    k = jax.lax.dot_general(wk, diff, (((1,), (0,)), ((), ())),
                            preferred_element_type=jnp.float32)
    q = jax.lax.dot_general(wq, xb, (((1,), (0,)), ((), ())),
                            preferred_element_type=jnp.float32)
    qrep = jnp.concatenate([q] * _K, axis=1)
    prod = qrep * k
    e = jnp.sum(prod.reshape(_H, _DEPTH, _K * _NB), axis=1)
    e = e.reshape(_H, _K, _NB)
    et = jnp.transpose(e, (0, 2, 1))

    es = et * jnp.float32(_INV_SQRT_D)
    m = jnp.max(es, axis=-1, keepdims=True)
    p = jnp.exp2((es - m) * jnp.float32(_LOG2E))
    s = jnp.sum(p, axis=-1, keepdims=True)
    attn = p * pl.reciprocal(s, approx=True)
    attn_t = jnp.transpose(attn, (0, 2, 1))
    mean_t = jnp.sum(attn_t, axis=1, keepdims=True) * jnp.float32(_INV_K)
    dev_t = attn_t - mean_t
    var = jnp.sum(dev_t * dev_t, axis=1) * jnp.float32(_INV_K)
    aps = jnp.sqrt(var)
    attn_ref[0] = attn
    aps_ref[0] = aps


def _attention(x, idx_t, Wq, Wk):
    return pl.pallas_call(
        _attn_kernel,
        grid=(_B, _N // _NB),
        in_specs=[
            pl.BlockSpec((1, _C, _NB), lambda b, j: (b, 0, j)),
            pl.BlockSpec((1, _C, _N), lambda b, j: (b, 0, 0)),
            pl.BlockSpec((1, _K, _NB), lambda b, j: (b, 0, j)),
            pl.BlockSpec((_C, _C), lambda b, j: (0, 0)),
            pl.BlockSpec((_C, _C), lambda b, j: (0, 0)),
        ],
        out_specs=[
            pl.BlockSpec((1, _H, _NB, _K), lambda b, j: (b, 0, j, 0)),
            pl.BlockSpec((1, _H, _NB), lambda b, j: (b, 0, j)),
        ],
        out_shape=[
            jax.ShapeDtypeStruct((_B, _H, _N, _K), jnp.float32),
            jax.ShapeDtypeStruct((_B, _H, _N), jnp.float32),
        ],
    )(x, x, idx_t, Wq, Wk)


def _split_heads(t):
    b, c, n, l = t.shape
    t = t.reshape(b, _H, c // _H, n, l)
    return jnp.transpose(t, (0, 1, 3, 4, 2))


def kernel(x, Wq, Wk, Wv):
    B, C, N, K, M, H = _B, _C, _N, _K, _M, _H
    DEPTH = _DEPTH
    idx_nn = _knn(x)
    idx_t = jnp.transpose(idx_nn, (0, 2, 1)).astype(jnp.int32)
    attn_bhnk, aps = _attention(x, idx_t, Wq, Wk)
    attn = attn_bhnk[:, :, :, None, :]

    neigh = jax.vmap(lambda xb, ib: xb[:, ib])(x, idx_nn)
    diff = neigh - x[:, :, :, None]
    v = jnp.einsum('oc,bcnk->bonk', Wv, diff)
    v = _split_heads(v)
    _, idx_top = jax.lax.top_k(aps, M)
    _, idx_drop = jax.lax.top_k(-aps, N - M)

    def _gather(att, vv, idx, m):
        ia = jnp.broadcast_to(idx[:, :, :, None, None], (B, H, m, 1, K))
        a_sel = jnp.take_along_axis(att, ia, axis=2)
        iv = jnp.broadcast_to(idx[:, :, :, None, None], (B, H, m, K, DEPTH))
        v_sel = jnp.take_along_axis(vv, iv, axis=2)
        out = jnp.einsum('bhmlk,bhmkd->bhmld', a_sel, v_sel)[:, :, :, 0, :]
        out = jnp.transpose(out, (0, 2, 1, 3)).reshape(B, m, H * DEPTH)
        return jnp.transpose(out, (0, 2, 1))

    x_ds = _gather(attn, v, idx_top, M)
    x_drop = _gather(attn, v, idx_drop, N - M)
    return ((x_ds, idx_top), (x_drop, idx_drop))


# speedup vs baseline: 13.9261x; 13.2499x over previous
import jax
import jax.numpy as jnp
import numpy as np
from jax.experimental import pallas as pl

_B, _C, _N, _K, _M, _H = 4, 128, 2048, 32, 1024, 4
_DEPTH = _C // _H
_RB = 256
_NB = 128


def _pairwise_kernel(xr_ref, xf_ref, o_ref):
    xr = xr_ref[0]
    xf = xf_ref[0]
    d = jax.lax.dot_general(xr, xf, (((0,), (0,)), ((), ())),
                            preferred_element_type=jnp.float32)
    inner = -2.0 * d
    xx_f = jnp.sum(xf * xf, axis=0, keepdims=True)
    xx_r = jnp.sum(xr * xr, axis=0, keepdims=True)
    t1 = -xx_r.reshape(_RB, 1)
    o_ref[0] = (t1 - inner) - xx_f


def _pairwise(x):
    return pl.pallas_call(
        _pairwise_kernel,
        grid=(_B, _N // _RB),
        in_specs=[
            pl.BlockSpec((1, _C, _RB), lambda b, j: (b, 0, j)),
            pl.BlockSpec((1, _C, _N), lambda b, j: (b, 0, 0)),
        ],
        out_specs=pl.BlockSpec((1, _RB, _N), lambda b, j: (b, j, 0)),
        out_shape=jax.ShapeDtypeStruct((_B, _N, _N), jnp.float32),
    )(x, x)


def _knn_kernel(xr_ref, xf_ref, idx_ref):
    xr = xr_ref[0]
    xf = xf_ref[0]
    d = jax.lax.dot_general(xr, xf, (((0,), (0,)), ((), ())),
                            preferred_element_type=jnp.float32)
    inner = -2.0 * d
    xx_f = jnp.sum(xf * xf, axis=0, keepdims=True)
    xx_r = jnp.sum(xr * xr, axis=0, keepdims=True)
    t1 = -xx_r.reshape(_RB, 1)
    p = (t1 - inner) - xx_f

    lane = jax.lax.broadcasted_iota(jnp.int32, (_RB, _N), 1)
    col = jax.lax.broadcasted_iota(jnp.int32, (_RB, _K), 1)

    def body(kk, carry):
        p, acc = carry
        m = jnp.max(p, axis=1, keepdims=True)
        cand = jnp.where(p == m, lane, _N)
        amin = jnp.min(cand, axis=1, keepdims=True)
        acc = jnp.where(col == kk, amin, acc)
        p = jnp.where(lane == amin, -jnp.inf, p)
        return p, acc

    _, acc = jax.lax.fori_loop(
        0, _K, body, (p, jnp.zeros((_RB, _K), jnp.int32)))
    idx_ref[0] = acc


def _knn(x):
    return pl.pallas_call(
        _knn_kernel,
        grid=(_B, _N // _RB),
        in_specs=[
            pl.BlockSpec((1, _C, _RB), lambda b, j: (b, 0, j)),
            pl.BlockSpec((1, _C, _N), lambda b, j: (b, 0, 0)),
        ],
        out_specs=pl.BlockSpec((1, _RB, _K), lambda b, j: (b, j, 0)),
        out_shape=jax.ShapeDtypeStruct((_B, _N, _K), jnp.int32),
    )(x, x)


_INV_SQRT_D = float(np.float32(1.0) / np.sqrt(np.float32(_DEPTH)))
_INV_K = float(np.float32(1.0) / np.float32(_K))
_LOG2E = float(np.float32(1.4426950408889634))


def _attn_kernel(xb_ref, xf_ref, idxt_ref, wq_ref, wk_ref, wv_ref,
                 out_ref, aps_ref):
    xb = xb_ref[0]
    xf = xf_ref[0]
    idxt = idxt_ref[0]
    wq = wq_ref[...]
    wk = wk_ref[...]

    slabs = []
    for kk in range(_K):
        ids = idxt[kk:kk + 1, :]
        row = jax.lax.broadcasted_iota(jnp.int32, (_N, _NB), 0)
        oh = jnp.where(row == ids, 1.0, 0.0).astype(jnp.float32)
        slabs.append(jax.lax.dot_general(
            xf, oh, (((1,), (0,)), ((), ())),
            precision=jax.lax.Precision.HIGHEST,
            preferred_element_type=jnp.float32))
    neigh = jnp.concatenate(slabs, axis=1)
    center = jnp.concatenate([xb] * _K, axis=1)
    diff = neigh - center

    k = jax.lax.dot_general(wk, diff, (((1,), (0,)), ((), ())),
                            preferred_element_type=jnp.float32)
    q = jax.lax.dot_general(wq, xb, (((1,), (0,)), ((), ())),
                            preferred_element_type=jnp.float32)
    qrep = jnp.concatenate([q] * _K, axis=1)
    prod = qrep * k
    e = jnp.sum(prod.reshape(_H, _DEPTH, _K * _NB), axis=1)
    e = e.reshape(_H, _K, _NB)
    et = jnp.transpose(e, (0, 2, 1))

    es = et * jnp.float32(_INV_SQRT_D)
    m = jnp.max(es, axis=-1, keepdims=True)
    p = jnp.exp2((es - m) * jnp.float32(_LOG2E))
    s = jnp.sum(p, axis=-1, keepdims=True)
    attn = p * pl.reciprocal(s, approx=True)
    attn_t = jnp.transpose(attn, (0, 2, 1))
    mean_t = jnp.sum(attn_t, axis=1, keepdims=True) * jnp.float32(_INV_K)
    dev_t = attn_t - mean_t
    var = jnp.sum(dev_t * dev_t, axis=1) * jnp.float32(_INV_K)
    aps = jnp.sqrt(var)
    aps_ref[0] = aps

    wv = wv_ref[...]
    v = jax.lax.dot_general(wv, diff, (((1,), (0,)), ((), ())),
                            preferred_element_type=jnp.float32)
    out = jnp.zeros((_C, _NB), jnp.float32)
    for kk in range(_K):
        a_kk = attn_t[:, kk:kk + 1, :]
        a_b = jnp.broadcast_to(a_kk, (_H, _DEPTH, _NB)).reshape(_C, _NB)
        out = out + a_b * v[:, kk * _NB:(kk + 1) * _NB]
    out_ref[0] = out


def _attention(x, idx_t, Wq, Wk, Wv):
    return pl.pallas_call(
        _attn_kernel,
        grid=(_B, _N // _NB),
        in_specs=[
            pl.BlockSpec((1, _C, _NB), lambda b, j: (b, 0, j)),
            pl.BlockSpec((1, _C, _N), lambda b, j: (b, 0, 0)),
            pl.BlockSpec((1, _K, _NB), lambda b, j: (b, 0, j)),
            pl.BlockSpec((_C, _C), lambda b, j: (0, 0)),
            pl.BlockSpec((_C, _C), lambda b, j: (0, 0)),
            pl.BlockSpec((_C, _C), lambda b, j: (0, 0)),
        ],
        out_specs=[
            pl.BlockSpec((1, _C, _NB), lambda b, j: (b, 0, j)),
            pl.BlockSpec((1, _H, _NB), lambda b, j: (b, 0, j)),
        ],
        out_shape=[
            jax.ShapeDtypeStruct((_B, _C, _N), jnp.float32),
            jax.ShapeDtypeStruct((_B, _H, _N), jnp.float32),
        ],
    )(x, x, idx_t, Wq, Wk, Wv)


def _split_heads(t):
    b, c, n, l = t.shape
    t = t.reshape(b, _H, c // _H, n, l)
    return jnp.transpose(t, (0, 1, 3, 4, 2))


def kernel(x, Wq, Wk, Wv):
    B, C, N, K, M, H = _B, _C, _N, _K, _M, _H
    DEPTH = _DEPTH
    idx_nn = _knn(x)
    idx_t = jnp.transpose(idx_nn, (0, 2, 1)).astype(jnp.int32)
    out_full, aps = _attention(x, idx_t, Wq, Wk, Wv)
    _, idx_top = jax.lax.top_k(aps, M)
    _, idx_drop = jax.lax.top_k(-aps, N - M)

    out_h = out_full.reshape(B, H, DEPTH, N)

    def _gather(idx, m):
        ii = jnp.broadcast_to(idx[:, :, None, :], (B, H, DEPTH, m))
        sel = jnp.take_along_axis(out_h, ii, axis=3)
        return sel.reshape(B, C, m)

    x_ds = _gather(idx_top, M)
    x_drop = _gather(idx_drop, N - M)
    return ((x_ds, idx_top), (x_drop, idx_drop))
